# Initial kernel scaffold; baseline (speedup 1.0000x reference)
import functools

import jax
import jax.numpy as jnp
from jax import lax
from jax.experimental import pallas as pl
from jax.experimental.pallas import tpu as pltpu
from jax.experimental.pallas import tpu_sc as plsc

N = 10000
IN_F = 32
INPUT_SIZE = 4
OUT_F = 32
FH = 4
HID = 128
ED = 16
L5 = 5
E = 320000

NC = 2
NS = 16
NW = NC * NS
NPAD = 10240
EPAD = 327680
CHUNK = 128
EW = EPAD // NW
NCHUNKS = EW // CHUNK
SLC = NPAD // NS
WN = NPAD // NW
QNEG = -(1 << 30)
QCLIP = 100000.0

_f32 = jnp.float32
_i32 = jnp.int32
_u32 = jnp.uint32


def _mesh():
    return plsc.VectorSubcoreMesh(core_axis_name="c", subcore_axis_name="s")


def _wid():
    c = lax.axis_index("c")
    s = lax.axis_index("s")
    return c, s, s * NC + c


@functools.lru_cache(maxsize=None)
def _build_p1():
    @functools.partial(
        pl.kernel,
        mesh=_mesh(),
        out_type=[
            jax.ShapeDtypeStruct((NC, NPAD), _f32),
            jax.ShapeDtypeStruct((NC, NPAD, ED), _f32),
        ],
        scratch_types=[
            pltpu.VMEM((1, CHUNK), _i32),
            pltpu.VMEM((CHUNK, ED), _f32),
            pltpu.VMEM((1, CHUNK), _f32),
            pltpu.VMEM_SHARED((NPAD,), _f32),
            pltpu.VMEM_SHARED((NPAD, ED), _f32),
        ],
    )
    def p1(dst_hbm, ea_hbm, zea_hbm, zden_hbm, deg_out, easum_out,
           idxv, eav, ones, deg_sh, ea_sh):
        c, s, w = _wid()
        pltpu.sync_copy(zden_hbm, deg_sh.at[pl.ds(s * SLC, SLC)])
        pltpu.sync_copy(zea_hbm, ea_sh.at[pl.ds(s * SLC, SLC), :])
        for g in range(CHUNK // 16):
            ones[0, pl.ds(g * 16, 16)] = jnp.full((16,), 1.0, _f32)
        plsc.subcore_barrier()

        def body(i, carry):
            base = w * EW + i * CHUNK
            pltpu.sync_copy(dst_hbm.at[pl.ds(base, CHUNK)], idxv.at[0])
            pltpu.sync_copy(ea_hbm.at[pl.ds(base, CHUNK), :], eav)
            pltpu.sync_copy(eav, ea_sh.at[idxv.at[0]], add=True)
            pltpu.sync_copy(ones.at[0], deg_sh.at[idxv.at[0]], add=True)
            return carry

        lax.fori_loop(0, NCHUNKS, body, 0)
        plsc.subcore_barrier()
        pltpu.sync_copy(deg_sh.at[pl.ds(s * SLC, SLC)],
                        deg_out.at[c, pl.ds(s * SLC, SLC)])
        pltpu.sync_copy(ea_sh.at[pl.ds(s * SLC, SLC), :],
                        easum_out.at[c, pl.ds(s * SLC, SLC), :])

    return p1


@functools.lru_cache(maxsize=None)
def _build_sca():
    @functools.partial(
        pl.kernel,
        mesh=_mesh(),
        out_type=[
            jax.ShapeDtypeStruct((EPAD,), _f32),
            jax.ShapeDtypeStruct((NPAD,), _f32),
            jax.ShapeDtypeStruct((NC, NPAD), _i32),
        ],
        scratch_types=[
            pltpu.VMEM((NPAD,), _f32),
            pltpu.VMEM((NPAD,), _f32),
            pltpu.VMEM((NPAD,), _i32),
            pltpu.VMEM((CHUNK,), _i32),
            pltpu.VMEM((CHUNK,), _i32),
            pltpu.VMEM((CHUNK,), _f32),
            pltpu.VMEM((CHUNK,), _f32),
            pltpu.VMEM((17,), _i32),
            pltpu.VMEM((WN,), _f32),
            pltpu.VMEM((WN,), _f32),
            pltpu.VMEM((SLC,), _i32),
            pltpu.VMEM((SLC,), _i32),
            pltpu.VMEM_SHARED((NS, NPAD), _i32),
        ],
    )
    def sca(src_hbm, dst_hbm, ae_hbm, lae_hbm, asrc_hbm, adst_hbm, qneg_hbm,
            ee_out, el_out, qmax_out,
            asrc_t, adst_t, qmax_t, srcv, dstv, aev, ebuf, scr, laev, elv,
            racc, rtmp, qsh):
        c, s, w = _wid()
        pltpu.sync_copy(asrc_hbm, asrc_t)
        pltpu.sync_copy(adst_hbm, adst_t)
        pltpu.sync_copy(qneg_hbm, qmax_t)
        scr[16] = jnp.int32(-1)
        iota1 = lax.iota(_i32, 16) + 1

        def body(i, carry):
            base = w * EW + i * CHUNK
            pltpu.sync_copy(src_hbm.at[pl.ds(base, CHUNK)], srcv)
            pltpu.sync_copy(dst_hbm.at[pl.ds(base, CHUNK)], dstv)
            pltpu.sync_copy(ae_hbm.at[pl.ds(base, CHUNK)], aev)
            for g in range(CHUNK // 16):
                sl = pl.ds(g * 16, 16)
                s16 = srcv[sl]
                d16 = dstv[sl]
                a1 = plsc.load_gather(asrc_t, [s16])
                a2 = plsc.load_gather(adst_t, [d16])
                e = a1 + a2 + aev[sl]
                e = jnp.where(e >= 0.0, e, e * 0.2)
                ebuf[sl] = e
                q = jnp.clip(e, -QCLIP, QCLIP).astype(_i32)
                ku = (lax.bitcast_convert_type(d16, _u32) << 18) | \
                     lax.bitcast_convert_type(q + 131072, _u32)
                ks, _unused = plsc.sort_key_val(ku, ku)
                scr[pl.ds(0, 16)] = lax.bitcast_convert_type(ks, _i32)
                nxt = lax.bitcast_convert_type(
                    plsc.load_gather(scr, [iota1]), _u32)
                mask = (ks >> 18) != (nxt >> 18)
                dsort = lax.bitcast_convert_type(ks >> 18, _i32)
                qsort = lax.bitcast_convert_type(
                    ks & jnp.uint32(0x3FFFF), _i32) - 131072
                cur = plsc.load_gather(qmax_t, [dsort])
                plsc.store_scatter(qmax_t, [dsort],
                                   jnp.maximum(cur, qsort), mask=mask)
            pltpu.sync_copy(ebuf, ee_out.at[pl.ds(base, CHUNK)])
            return carry

        lax.fori_loop(0, NCHUNKS, body, 0)

        n0 = w * WN
        pltpu.sync_copy(lae_hbm.at[pl.ds(n0, WN)], laev)
        for g in range(WN // 16):
            sl16 = pl.ds(n0 + g * 16, 16)
            sl = pl.ds(g * 16, 16)
            e = asrc_t[sl16] + adst_t[sl16] + laev[sl]
            e = jnp.where(e >= 0.0, e, e * 0.2)
            elv[sl] = e
            q = jnp.clip(e, -QCLIP, QCLIP).astype(_i32)
            qmax_t[sl16] = jnp.maximum(qmax_t[sl16], q)
        pltpu.sync_copy(elv, el_out.at[pl.ds(n0, WN)])

        pltpu.sync_copy(qmax_t, qsh.at[s])
        plsc.subcore_barrier()
        pltpu.sync_copy(qsh.at[0, pl.ds(s * SLC, SLC)], racc)
        for r in range(1, NS):
            pltpu.sync_copy(qsh.at[r, pl.ds(s * SLC, SLC)], rtmp)
            for g in range(SLC // 16):
                sl = pl.ds(g * 16, 16)
                racc[sl] = jnp.maximum(racc[sl], rtmp[sl])
        pltpu.sync_copy(racc, qmax_out.at[c, pl.ds(s * SLC, SLC)])

    return sca


@functools.lru_cache(maxsize=None)
def _build_scbc():
    @functools.partial(
        pl.kernel,
        mesh=_mesh(),
        out_type=[
            jax.ShapeDtypeStruct((NC, NPAD, HID), _f32),
            jax.ShapeDtypeStruct((NC, NPAD), _f32),
        ],
        scratch_types=[
            pltpu.VMEM((NPAD,), _i32),
            pltpu.VMEM((NPAD,), _i32),
            pltpu.VMEM((CHUNK,), _i32),
            pltpu.VMEM((1, CHUNK), _i32),
            pltpu.VMEM((1, CHUNK), _i32),
            pltpu.VMEM((CHUNK,), _f32),
            pltpu.VMEM((CHUNK,), _f32),
            pltpu.VMEM((CHUNK, HID), _f32),
            pltpu.SemaphoreType.DMA,
            pltpu.VMEM_SHARED((NPAD, HID), _f32),
            pltpu.VMEM_SHARED((NPAD,), _f32),
        ],
    )
    def scbc(src_hbm, dst_hbm, ee_hbm, el_hbm, qmax_hbm, h_hbm,
             zacc_hbm, zden_hbm,
             p_out, den_out,
             bq_t, tq_t, srcv, dstv, linv, ev, exv, rows, sem, acc_sh, den_sh):
        c, s, w = _wid()
        pltpu.sync_copy(qmax_hbm.at[0], bq_t)
        pltpu.sync_copy(qmax_hbm.at[1], tq_t)
        for g in range(NPAD // 16):
            sl = pl.ds(g * 16, 16)
            bq_t[sl] = jnp.maximum(bq_t[sl], tq_t[sl])
        pltpu.sync_copy(zacc_hbm, acc_sh.at[pl.ds(s * SLC, SLC), :])
        pltpu.sync_copy(zden_hbm, den_sh.at[pl.ds(s * SLC, SLC)])
        plsc.subcore_barrier()

        def scale_rows():
            def rbody(r, carry):
                xr = exv[r]
                for k in range(HID // 16):
                    sl = pl.ds(k * 16, 16)
                    rows[r, sl] = rows[r, sl] * xr
                return carry
            lax.fori_loop(0, CHUNK, rbody, 0)

        for k in range(3):
            j = k * NW + w

            @pl.when(j < NPAD // CHUNK)
            def _():
                i0 = j * CHUNK
                pltpu.sync_copy(el_hbm.at[pl.ds(i0, CHUNK)], ev)
                pltpu.sync_copy(h_hbm.at[pl.ds(i0, CHUNK), :], rows)
                for g in range(CHUNK // 16):
                    sl = pl.ds(g * 16, 16)
                    b16 = bq_t[pl.ds(i0 + g * 16, 16)].astype(_f32)
                    exv[sl] = jnp.exp(ev[sl] - b16)
                    linv[0, sl] = lax.iota(_i32, 16) + (i0 + g * 16)
                scale_rows()
                pltpu.sync_copy(rows, acc_sh.at[linv.at[0]], add=True)
                pltpu.sync_copy(exv, den_sh.at[linv.at[0]], add=True)

        def body(i, carry):
            base = w * EW + i * CHUNK
            pltpu.sync_copy(src_hbm.at[pl.ds(base, CHUNK)], srcv)
            pltpu.sync_copy(dst_hbm.at[pl.ds(base, CHUNK)], dstv.at[0])
            pltpu.sync_copy(ee_hbm.at[pl.ds(base, CHUNK)], ev)
            pltpu.async_copy(h_hbm.at[srcv], rows, sem).wait()
            for g in range(CHUNK // 16):
                sl = pl.ds(g * 16, 16)
                d16 = dstv[0, sl]
                b16 = plsc.load_gather(bq_t, [d16]).astype(_f32)
                exv[sl] = jnp.exp(ev[sl] - b16)
            scale_rows()
            pltpu.sync_copy(rows, acc_sh.at[dstv.at[0]], add=True)
            pltpu.sync_copy(exv, den_sh.at[dstv.at[0]], add=True)
            return carry

        lax.fori_loop(0, NCHUNKS, body, 0)
        plsc.subcore_barrier()
        pltpu.sync_copy(acc_sh.at[pl.ds(s * SLC, SLC), :],
                        p_out.at[c, pl.ds(s * SLC, SLC), :])
        pltpu.sync_copy(den_sh.at[pl.ds(s * SLC, SLC)],
                        den_out.at[c, pl.ds(s * SLC, SLC)])

    return scbc


_DOT = dict(precision=lax.Precision.HIGHEST, preferred_element_type=_f32)


def _wae8(we, ate):
    wae = lax.dot_general(we, ate, (((2,), (1,)), ((0,), (0,))), **_DOT)
    return jnp.concatenate([wae, jnp.zeros((8 - L5, ED), _f32)], axis=0)


def _aek_body(ea_ref, we_ref, ate_ref, out_ref):
    wae = _wae8(we_ref[...], ate_ref[...])
    out_ref[...] = lax.dot_general(wae, ea_ref[...], (((1,), (1,)), ((), ())),
                                   **_DOT)


def _aek(ea_p, We, att_e):
    be = 2048
    return pl.pallas_call(
        _aek_body,
        grid=(EPAD // be,),
        in_specs=[
            pl.BlockSpec((be, ED), lambda i: (i, 0)),
            pl.BlockSpec((L5, ED, HID), lambda i: (0, 0, 0)),
            pl.BlockSpec((L5, HID), lambda i: (0, 0)),
        ],
        out_specs=pl.BlockSpec((8, be), lambda i: (0, i)),
        out_shape=jax.ShapeDtypeStruct((8, EPAD), _f32),
    )(ea_p, We, att_e)


def _laek_body(easum_ref, deg_ref, we_ref, ate_ref, out_ref):
    wae = _wae8(we_ref[...], ate_ref[...])
    ea = easum_ref[0] + easum_ref[1]
    lae = lax.dot_general(wae, ea, (((1,), (1,)), ((), ())), **_DOT)
    deg = jnp.maximum(deg_ref[0] + deg_ref[1], 1.0)[None, :]
    out_ref[...] = lae / deg


def _laek(easum_part, deg_part, We, att_e):
    blk = 128
    return pl.pallas_call(
        _laek_body,
        grid=(NPAD // blk,),
        in_specs=[
            pl.BlockSpec((NC, blk, ED), lambda i: (0, i, 0)),
            pl.BlockSpec((NC, blk), lambda i: (0, i)),
            pl.BlockSpec((L5, ED, HID), lambda i: (0, 0, 0)),
            pl.BlockSpec((L5, HID), lambda i: (0, 0)),
        ],
        out_specs=pl.BlockSpec((8, blk), lambda i: (0, i)),
        out_shape=jax.ShapeDtypeStruct((8, NPAD), _f32),
    )(easum_part, deg_part, We, att_e)


def _head_tail(h, wg_ref, ats_ref, atd_ref, h_ref, as_ref, ad_ref):
    hn = jnp.dot(h, wg_ref[...], **_DOT)
    h_ref[...] = hn
    as_ref[...] = jnp.dot(hn, ats_ref[...], **_DOT)
    ad_ref[...] = jnp.dot(hn, atd_ref[...], **_DOT)


def _k0_body(x_ref, wemb_ref, bemb_ref, lng_ref, lnb_ref,
             wg_ref, ats_ref, atd_ref, h_ref, as_ref, ad_ref):
    h = jnp.dot(x_ref[...], wemb_ref[...], **_DOT) + bemb_ref[...][None, :]
    m = jnp.mean(h, axis=-1, keepdims=True)
    v = jnp.mean((h - m) ** 2, axis=-1, keepdims=True)
    h = (h - m) / jnp.sqrt(v + 1e-5) * lng_ref[...][None, :] \
        + lnb_ref[...][None, :]
    h = jnp.maximum(h, 0.0)
    _head_tail(h, wg_ref, ats_ref, atd_ref, h_ref, as_ref, ad_ref)


def _k0(x2, W_emb, b_emb, ln_g, ln_b, Wg0, ats0, atd0):
    blk = 128
    return pl.pallas_call(
        _k0_body,
        grid=(NPAD // blk,),
        in_specs=[
            pl.BlockSpec((blk, HID), lambda i: (i, 0)),
            pl.BlockSpec((HID, HID), lambda i: (0, 0)),
            pl.BlockSpec((HID,), lambda i: (0,)),
            pl.BlockSpec((HID,), lambda i: (0,)),
            pl.BlockSpec((HID,), lambda i: (0,)),
            pl.BlockSpec((HID, HID), lambda i: (0, 0)),
            pl.BlockSpec((HID, 1), lambda i: (0, 0)),
            pl.BlockSpec((HID, 1), lambda i: (0, 0)),
        ],
        out_specs=[
            pl.BlockSpec((blk, HID), lambda i: (i, 0)),
            pl.BlockSpec((blk, 1), lambda i: (i, 0)),
            pl.BlockSpec((blk, 1), lambda i: (i, 0)),
        ],
        out_shape=[
            jax.ShapeDtypeStruct((NPAD, HID), _f32),
            jax.ShapeDtypeStruct((NPAD, 1), _f32),
            jax.ShapeDtypeStruct((NPAD, 1), _f32),
        ],
    )(x2, W_emb, b_emb, ln_g, ln_b, Wg0, ats0, atd0)


def _finish(p_ref, d0_ref, d1_ref, bias_ref):
    p = p_ref[0] + p_ref[1]
    den = d0_ref[...] + d1_ref[...] + 1e-16
    return jnp.maximum(p / den + bias_ref[...][None, :], 0.0)


def _kl_body(p_ref, d0_ref, d1_ref, bias_ref,
             wg_ref, ats_ref, atd_ref, h_ref, as_ref, ad_ref):
    h = _finish(p_ref, d0_ref, d1_ref, bias_ref)
    _head_tail(h, wg_ref, ats_ref, atd_ref, h_ref, as_ref, ad_ref)


def _kl(p_part, d0, d1, bias, Wg_l, ats, atd):
    blk = 128
    return pl.pallas_call(
        _kl_body,
        grid=(NPAD // blk,),
        in_specs=[
            pl.BlockSpec((NC, blk, HID), lambda i: (0, i, 0)),
            pl.BlockSpec((blk, 1), lambda i: (i, 0)),
            pl.BlockSpec((blk, 1), lambda i: (i, 0)),
            pl.BlockSpec((HID,), lambda i: (0,)),
            pl.BlockSpec((HID, HID), lambda i: (0, 0)),
            pl.BlockSpec((HID, 1), lambda i: (0, 0)),
            pl.BlockSpec((HID, 1), lambda i: (0, 0)),
        ],
        out_specs=[
            pl.BlockSpec((blk, HID), lambda i: (i, 0)),
            pl.BlockSpec((blk, 1), lambda i: (i, 0)),
            pl.BlockSpec((blk, 1), lambda i: (i, 0)),
        ],
        out_shape=[
            jax.ShapeDtypeStruct((NPAD, HID), _f32),
            jax.ShapeDtypeStruct((NPAD, 1), _f32),
            jax.ShapeDtypeStruct((NPAD, 1), _f32),
        ],
    )(p_part, d0, d1, bias, Wg_l, ats, atd)


def _kdec_body(p_ref, d0_ref, d1_ref, bias_ref, wdec_ref, bdec_ref, out_ref):
    h = _finish(p_ref, d0_ref, d1_ref, bias_ref)
    out_ref[...] = jnp.dot(h, wdec_ref[...], **_DOT) + bdec_ref[...][None, :]


def _kdec(p_part, d0, d1, bias, W_dec, b_dec):
    blk = 128
    return pl.pallas_call(
        _kdec_body,
        grid=(NPAD // blk,),
        in_specs=[
            pl.BlockSpec((NC, blk, HID), lambda i: (0, i, 0)),
            pl.BlockSpec((blk, 1), lambda i: (i, 0)),
            pl.BlockSpec((blk, 1), lambda i: (i, 0)),
            pl.BlockSpec((HID,), lambda i: (0,)),
            pl.BlockSpec((HID, OUT_F * FH), lambda i: (0, 0)),
            pl.BlockSpec((OUT_F * FH,), lambda i: (0,)),
        ],
        out_specs=pl.BlockSpec((blk, OUT_F * FH), lambda i: (i, 0)),
        out_shape=jax.ShapeDtypeStruct((NPAD, OUT_F * FH), _f32),
    )(p_part, d0, d1, bias, W_dec, b_dec)


def kernel(x, edge_index, edge_attr, W_emb, b_emb, ln_g, ln_b,
           Wg, att_s, att_d, We, att_e, bg, W_dec, b_dec):
    src = edge_index[0]
    dst = edge_index[1]
    npe = EPAD - E
    pad_idx = (jnp.arange(npe, dtype=_i32) % (NPAD - N)) + N
    src_p = jnp.concatenate([src, pad_idx])
    dst_p = jnp.concatenate([dst, pad_idx])
    ea_p = jnp.concatenate([edge_attr, jnp.zeros((npe, ED), _f32)], axis=0)
    x2 = jnp.pad(x.reshape(N, IN_F * INPUT_SIZE), ((0, NPAD - N), (0, 0)))

    z_acc = jnp.zeros((SLC, HID), _f32)
    z_den = jnp.zeros((SLC,), _f32)
    z_ea = jnp.zeros((SLC, ED), _f32)
    qneg = jnp.full((NPAD,), QNEG, _i32)

    deg_part, easum_part = _build_p1()(dst_p, ea_p, z_ea, z_den)
    AE = _aek(ea_p, We, att_e)
    LAE = _laek(easum_part, deg_part, We, att_e)

    H, a_s, a_d = _k0(x2, W_emb, b_emb, ln_g, ln_b,
                      Wg[0], att_s[0][:, None], att_d[0][:, None])
    out = None
    for l in range(L5):
        asrc = a_s[:, 0]
        adst = a_d[:, 0]
        ee, el, qmax_part = _build_sca()(
            src_p, dst_p, AE[l], LAE[l], asrc, adst, qneg)
        p_part, den_part = _build_scbc()(
            src_p, dst_p, ee, el, qmax_part, H, z_acc, z_den)
        d0 = den_part[0][:, None]
        d1 = den_part[1][:, None]
        if l < L5 - 1:
            H, a_s, a_d = _kl(p_part, d0, d1, bg[l],
                              Wg[l + 1], att_s[l + 1][:, None],
                              att_d[l + 1][:, None])
        else:
            out = _kdec(p_part, d0, d1, bg[l], W_dec, b_dec)
    return out[:N].reshape(N, OUT_F, FH)


# speedup vs baseline: 13.3921x; 13.3921x over previous
import functools

import jax
import jax.numpy as jnp
from jax import lax
from jax.experimental import pallas as pl
from jax.experimental.pallas import tpu as pltpu
from jax.experimental.pallas import tpu_sc as plsc

N = 10000
IN_F = 32
INPUT_SIZE = 4
OUT_F = 32
FH = 4
HID = 128
ED = 16
L5 = 5
E = 320000

NC = 2
NS = 16
NW = NC * NS
NPAD = 10240
EPAD = 327680
CHUNK = 128
EW = EPAD // NW
NCHUNKS = EW // CHUNK
SLC = NPAD // NS
WN = NPAD // NW
QNEG = -(1 << 30)
QCLIP = 100000.0

_f32 = jnp.float32
_i32 = jnp.int32
_u32 = jnp.uint32


def _mesh():
    return plsc.VectorSubcoreMesh(core_axis_name="c", subcore_axis_name="s")


def _wid():
    c = lax.axis_index("c")
    s = lax.axis_index("s")
    return c, s, s * NC + c


@functools.lru_cache(maxsize=None)
def _build_p1():
    @functools.partial(
        pl.kernel,
        mesh=_mesh(),
        compiler_params=pltpu.CompilerParams(needs_layout_passes=False),
        out_type=[
            jax.ShapeDtypeStruct((NC, NPAD), _f32),
            jax.ShapeDtypeStruct((NC, NPAD, ED), _f32),
        ],
        scratch_types=[
            pltpu.VMEM((1, CHUNK), _i32),
            pltpu.VMEM((CHUNK, ED), _f32),
            pltpu.VMEM((1, CHUNK), _f32),
            pltpu.VMEM_SHARED((NPAD,), _f32),
            pltpu.VMEM_SHARED((NPAD, ED), _f32),
        ],
    )
    def p1(dst_hbm, ea_hbm, zea_hbm, zden_hbm, deg_out, easum_out,
           idxv, eav, ones, deg_sh, ea_sh):
        c, s, w = _wid()
        pltpu.sync_copy(zden_hbm, deg_sh.at[pl.ds(s * SLC, SLC)])
        pltpu.sync_copy(zea_hbm, ea_sh.at[pl.ds(s * SLC, SLC), :])
        for g in range(CHUNK // 16):
            ones[0, pl.ds(g * 16, 16)] = jnp.full((16,), 1.0, _f32)
        plsc.subcore_barrier()

        def body(i, carry):
            base = w * EW + i * CHUNK
            pltpu.sync_copy(dst_hbm.at[pl.ds(base, CHUNK)], idxv.at[0])
            pltpu.sync_copy(ea_hbm.at[pl.ds(base, CHUNK), :], eav)
            pltpu.sync_copy(eav, ea_sh.at[idxv.at[0]], add=True)
            pltpu.sync_copy(ones.at[0], deg_sh.at[idxv.at[0]], add=True)
            return carry

        lax.fori_loop(0, NCHUNKS, body, 0)
        plsc.subcore_barrier()
        pltpu.sync_copy(deg_sh.at[pl.ds(s * SLC, SLC)],
                        deg_out.at[c, pl.ds(s * SLC, SLC)])
        pltpu.sync_copy(ea_sh.at[pl.ds(s * SLC, SLC), :],
                        easum_out.at[c, pl.ds(s * SLC, SLC), :])

    return p1


@functools.lru_cache(maxsize=None)
def _build_sca():
    @functools.partial(
        pl.kernel,
        mesh=_mesh(),
        compiler_params=pltpu.CompilerParams(needs_layout_passes=False),
        out_type=[
            jax.ShapeDtypeStruct((EPAD,), _f32),
            jax.ShapeDtypeStruct((NPAD,), _f32),
            jax.ShapeDtypeStruct((NC, NPAD), _i32),
        ],
        scratch_types=[
            pltpu.VMEM((NPAD,), _f32),
            pltpu.VMEM((NPAD,), _f32),
            pltpu.VMEM((NPAD,), _i32),
            pltpu.VMEM((CHUNK,), _i32),
            pltpu.VMEM((CHUNK,), _i32),
            pltpu.VMEM((CHUNK,), _f32),
            pltpu.VMEM((CHUNK,), _f32),
            pltpu.VMEM((32,), _i32),
            pltpu.VMEM((WN,), _f32),
            pltpu.VMEM((WN,), _f32),
            pltpu.VMEM((SLC,), _i32),
            pltpu.VMEM((SLC,), _i32),
            pltpu.VMEM_SHARED((NS, NPAD), _i32),
        ],
    )
    def sca(src_hbm, dst_hbm, ae_hbm, lae_hbm, asrc_hbm, adst_hbm, qneg_hbm,
            ee_out, el_out, qmax_out,
            asrc_t, adst_t, qmax_t, srcv, dstv, aev, ebuf, scr, laev, elv,
            racc, rtmp, qsh):
        c, s, w = _wid()
        pltpu.sync_copy(asrc_hbm, asrc_t)
        pltpu.sync_copy(adst_hbm, adst_t)
        pltpu.sync_copy(qneg_hbm, qmax_t)
        scr[pl.ds(16, 16)] = jnp.full((16,), -1, _i32)
        iota1 = lax.iota(_i32, 16) + 1

        def body(i, carry):
            base = w * EW + i * CHUNK
            pltpu.sync_copy(src_hbm.at[pl.ds(base, CHUNK)], srcv)
            pltpu.sync_copy(dst_hbm.at[pl.ds(base, CHUNK)], dstv)
            pltpu.sync_copy(ae_hbm.at[pl.ds(base, CHUNK)], aev)
            for g in range(CHUNK // 16):
                sl = pl.ds(g * 16, 16)
                s16 = srcv[sl]
                d16 = dstv[sl]
                a1 = plsc.load_gather(asrc_t, [s16])
                a2 = plsc.load_gather(adst_t, [d16])
                e = a1 + a2 + aev[sl]
                e = jnp.where(e >= 0.0, e, e * 0.2)
                ebuf[sl] = e
                q = jnp.clip(e, -QCLIP, QCLIP).astype(_i32)
                ku = (lax.bitcast_convert_type(d16, _u32) << 18) | \
                     lax.bitcast_convert_type(q + 131072, _u32)
                ks, _unused = plsc.sort_key_val(ku, ku)
                scr[pl.ds(0, 16)] = lax.bitcast_convert_type(ks, _i32)
                nxt = lax.bitcast_convert_type(
                    plsc.load_gather(scr, [iota1]), _u32)
                mask = (ks >> 18) != (nxt >> 18)
                dsort = lax.bitcast_convert_type(ks >> 18, _i32)
                qsort = lax.bitcast_convert_type(
                    ks & jnp.uint32(0x3FFFF), _i32) - 131072
                cur = plsc.load_gather(qmax_t, [dsort])
                plsc.store_scatter(qmax_t, [dsort],
                                   jnp.maximum(cur, qsort), mask=mask)
            pltpu.sync_copy(ebuf, ee_out.at[pl.ds(base, CHUNK)])
            return carry

        lax.fori_loop(0, NCHUNKS, body, 0)

        n0 = w * WN
        pltpu.sync_copy(lae_hbm.at[pl.ds(n0, WN)], laev)
        for g in range(WN // 16):
            sl16 = pl.ds(n0 + g * 16, 16)
            sl = pl.ds(g * 16, 16)
            e = asrc_t[sl16] + adst_t[sl16] + laev[sl]
            e = jnp.where(e >= 0.0, e, e * 0.2)
            elv[sl] = e
            q = jnp.clip(e, -QCLIP, QCLIP).astype(_i32)
            qmax_t[sl16] = jnp.maximum(qmax_t[sl16], q)
        pltpu.sync_copy(elv, el_out.at[pl.ds(n0, WN)])

        pltpu.sync_copy(qmax_t, qsh.at[s])
        plsc.subcore_barrier()
        pltpu.sync_copy(qsh.at[0, pl.ds(s * SLC, SLC)], racc)
        for r in range(1, NS):
            pltpu.sync_copy(qsh.at[r, pl.ds(s * SLC, SLC)], rtmp)
            for g in range(SLC // 16):
                sl = pl.ds(g * 16, 16)
                racc[sl] = jnp.maximum(racc[sl], rtmp[sl])
        pltpu.sync_copy(racc, qmax_out.at[c, pl.ds(s * SLC, SLC)])

    return sca


@functools.lru_cache(maxsize=None)
def _build_scbc():
    @functools.partial(
        pl.kernel,
        mesh=_mesh(),
        compiler_params=pltpu.CompilerParams(needs_layout_passes=False),
        out_type=[
            jax.ShapeDtypeStruct((NC, NPAD, HID), _f32),
            jax.ShapeDtypeStruct((NC, NPAD), _f32),
        ],
        scratch_types=[
            pltpu.VMEM((NPAD,), _i32),
            pltpu.VMEM((NPAD,), _i32),
            pltpu.VMEM((CHUNK,), _i32),
            pltpu.VMEM((1, CHUNK), _i32),
            pltpu.VMEM((1, CHUNK), _i32),
            pltpu.VMEM((CHUNK,), _f32),
            pltpu.VMEM((CHUNK,), _f32),
            pltpu.VMEM((CHUNK, HID), _f32),
            pltpu.SemaphoreType.DMA,
            pltpu.VMEM_SHARED((NPAD, HID), _f32),
            pltpu.VMEM_SHARED((NPAD,), _f32),
        ],
    )
    def scbc(src_hbm, dst_hbm, ee_hbm, el_hbm, qmax_hbm, h_hbm,
             zacc_hbm, zden_hbm,
             p_out, den_out,
             bq_t, tq_t, srcv, dstv, linv, ev, exv, rows, sem, acc_sh, den_sh):
        c, s, w = _wid()
        pltpu.sync_copy(qmax_hbm.at[0], bq_t)
        pltpu.sync_copy(qmax_hbm.at[1], tq_t)
        for g in range(NPAD // 16):
            sl = pl.ds(g * 16, 16)
            bq_t[sl] = jnp.maximum(bq_t[sl], tq_t[sl])
        pltpu.sync_copy(zacc_hbm, acc_sh.at[pl.ds(s * SLC, SLC), :])
        pltpu.sync_copy(zden_hbm, den_sh.at[pl.ds(s * SLC, SLC)])
        plsc.subcore_barrier()

        def scale_rows():
            def gbody(g, carry):
                ex16 = exv[pl.ds(g * 16, 16)]
                base = g * 16
                for rr in range(16):
                    xr = ex16[rr]
                    r = base + rr
                    for k in range(HID // 16):
                        sl = pl.ds(k * 16, 16)
                        rows[r, sl] = rows[r, sl] * xr
                return carry
            lax.fori_loop(0, CHUNK // 16, gbody, 0)

        for k in range(3):
            j = k * NW + w

            @pl.when(j < NPAD // CHUNK)
            def _():
                i0 = j * CHUNK
                pltpu.sync_copy(el_hbm.at[pl.ds(i0, CHUNK)], ev)
                pltpu.sync_copy(h_hbm.at[pl.ds(i0, CHUNK), :], rows)
                for g in range(CHUNK // 16):
                    sl = pl.ds(g * 16, 16)
                    b16 = bq_t[pl.ds(i0 + g * 16, 16)].astype(_f32)
                    exv[sl] = jnp.exp(ev[sl] - b16)
                    linv[0, sl] = lax.iota(_i32, 16) + (i0 + g * 16)
                scale_rows()
                pltpu.sync_copy(rows, acc_sh.at[linv.at[0]], add=True)
                pltpu.sync_copy(exv, den_sh.at[linv.at[0]], add=True)

        def body(i, carry):
            base = w * EW + i * CHUNK
            pltpu.sync_copy(src_hbm.at[pl.ds(base, CHUNK)], srcv)
            pltpu.sync_copy(dst_hbm.at[pl.ds(base, CHUNK)], dstv.at[0])
            pltpu.sync_copy(ee_hbm.at[pl.ds(base, CHUNK)], ev)
            pltpu.async_copy(h_hbm.at[srcv], rows, sem).wait()
            for g in range(CHUNK // 16):
                sl = pl.ds(g * 16, 16)
                d16 = dstv[0, sl]
                b16 = plsc.load_gather(bq_t, [d16]).astype(_f32)
                exv[sl] = jnp.exp(ev[sl] - b16)
            scale_rows()
            pltpu.sync_copy(rows, acc_sh.at[dstv.at[0]], add=True)
            pltpu.sync_copy(exv, den_sh.at[dstv.at[0]], add=True)
            return carry

        lax.fori_loop(0, NCHUNKS, body, 0)
        plsc.subcore_barrier()
        pltpu.sync_copy(acc_sh.at[pl.ds(s * SLC, SLC), :],
                        p_out.at[c, pl.ds(s * SLC, SLC), :])
        pltpu.sync_copy(den_sh.at[pl.ds(s * SLC, SLC)],
                        den_out.at[c, pl.ds(s * SLC, SLC)])

    return scbc


_DOT = dict(precision=lax.Precision.HIGHEST, preferred_element_type=_f32)


def _wae8(we, ate):
    wae = lax.dot_general(we, ate, (((2,), (1,)), ((0,), (0,))), **_DOT)
    return jnp.concatenate([wae, jnp.zeros((8 - L5, ED), _f32)], axis=0)


def _aek_body(ea_ref, we_ref, ate_ref, out_ref):
    wae = _wae8(we_ref[...], ate_ref[...])
    out_ref[...] = lax.dot_general(wae, ea_ref[...], (((1,), (1,)), ((), ())),
                                   **_DOT)


def _aek(ea_p, We, att_e):
    be = 2048
    return pl.pallas_call(
        _aek_body,
        grid=(EPAD // be,),
        in_specs=[
            pl.BlockSpec((be, ED), lambda i: (i, 0)),
            pl.BlockSpec((L5, ED, HID), lambda i: (0, 0, 0)),
            pl.BlockSpec((L5, HID), lambda i: (0, 0)),
        ],
        out_specs=pl.BlockSpec((8, be), lambda i: (0, i)),
        out_shape=jax.ShapeDtypeStruct((8, EPAD), _f32),
    )(ea_p, We, att_e)


def _laek_body(easum_ref, deg_ref, we_ref, ate_ref, out_ref):
    wae = _wae8(we_ref[...], ate_ref[...])
    ea = easum_ref[0] + easum_ref[1]
    lae = lax.dot_general(wae, ea, (((1,), (1,)), ((), ())), **_DOT)
    deg = jnp.maximum(deg_ref[0] + deg_ref[1], 1.0)[None, :]
    out_ref[...] = lae / deg


def _laek(easum_part, deg_part, We, att_e):
    blk = 128
    return pl.pallas_call(
        _laek_body,
        grid=(NPAD // blk,),
        in_specs=[
            pl.BlockSpec((NC, blk, ED), lambda i: (0, i, 0)),
            pl.BlockSpec((NC, blk), lambda i: (0, i)),
            pl.BlockSpec((L5, ED, HID), lambda i: (0, 0, 0)),
            pl.BlockSpec((L5, HID), lambda i: (0, 0)),
        ],
        out_specs=pl.BlockSpec((8, blk), lambda i: (0, i)),
        out_shape=jax.ShapeDtypeStruct((8, NPAD), _f32),
    )(easum_part, deg_part, We, att_e)


def _head_tail(h, wg_ref, ats_ref, atd_ref, h_ref, as_ref, ad_ref):
    hn = jnp.dot(h, wg_ref[...], **_DOT)
    h_ref[...] = hn
    as_ref[...] = jnp.dot(hn, ats_ref[...], **_DOT)
    ad_ref[...] = jnp.dot(hn, atd_ref[...], **_DOT)


def _k0_body(x_ref, wemb_ref, bemb_ref, lng_ref, lnb_ref,
             wg_ref, ats_ref, atd_ref, h_ref, as_ref, ad_ref):
    h = jnp.dot(x_ref[...], wemb_ref[...], **_DOT) + bemb_ref[...][None, :]
    m = jnp.mean(h, axis=-1, keepdims=True)
    v = jnp.mean((h - m) ** 2, axis=-1, keepdims=True)
    h = (h - m) / jnp.sqrt(v + 1e-5) * lng_ref[...][None, :] \
        + lnb_ref[...][None, :]
    h = jnp.maximum(h, 0.0)
    _head_tail(h, wg_ref, ats_ref, atd_ref, h_ref, as_ref, ad_ref)


def _k0(x2, W_emb, b_emb, ln_g, ln_b, Wg0, ats0, atd0):
    blk = 128
    return pl.pallas_call(
        _k0_body,
        grid=(NPAD // blk,),
        in_specs=[
            pl.BlockSpec((blk, HID), lambda i: (i, 0)),
            pl.BlockSpec((HID, HID), lambda i: (0, 0)),
            pl.BlockSpec((HID,), lambda i: (0,)),
            pl.BlockSpec((HID,), lambda i: (0,)),
            pl.BlockSpec((HID,), lambda i: (0,)),
            pl.BlockSpec((HID, HID), lambda i: (0, 0)),
            pl.BlockSpec((HID, 1), lambda i: (0, 0)),
            pl.BlockSpec((HID, 1), lambda i: (0, 0)),
        ],
        out_specs=[
            pl.BlockSpec((blk, HID), lambda i: (i, 0)),
            pl.BlockSpec((blk, 1), lambda i: (i, 0)),
            pl.BlockSpec((blk, 1), lambda i: (i, 0)),
        ],
        out_shape=[
            jax.ShapeDtypeStruct((NPAD, HID), _f32),
            jax.ShapeDtypeStruct((NPAD, 1), _f32),
            jax.ShapeDtypeStruct((NPAD, 1), _f32),
        ],
    )(x2, W_emb, b_emb, ln_g, ln_b, Wg0, ats0, atd0)


def _finish(p_ref, d0_ref, d1_ref, bias_ref):
    p = p_ref[0] + p_ref[1]
    den = d0_ref[...] + d1_ref[...] + 1e-16
    return jnp.maximum(p / den + bias_ref[...][None, :], 0.0)


def _kl_body(p_ref, d0_ref, d1_ref, bias_ref,
             wg_ref, ats_ref, atd_ref, h_ref, as_ref, ad_ref):
    h = _finish(p_ref, d0_ref, d1_ref, bias_ref)
    _head_tail(h, wg_ref, ats_ref, atd_ref, h_ref, as_ref, ad_ref)


def _kl(p_part, d0, d1, bias, Wg_l, ats, atd):
    blk = 128
    return pl.pallas_call(
        _kl_body,
        grid=(NPAD // blk,),
        in_specs=[
            pl.BlockSpec((NC, blk, HID), lambda i: (0, i, 0)),
            pl.BlockSpec((blk, 1), lambda i: (i, 0)),
            pl.BlockSpec((blk, 1), lambda i: (i, 0)),
            pl.BlockSpec((HID,), lambda i: (0,)),
            pl.BlockSpec((HID, HID), lambda i: (0, 0)),
            pl.BlockSpec((HID, 1), lambda i: (0, 0)),
            pl.BlockSpec((HID, 1), lambda i: (0, 0)),
        ],
        out_specs=[
            pl.BlockSpec((blk, HID), lambda i: (i, 0)),
            pl.BlockSpec((blk, 1), lambda i: (i, 0)),
            pl.BlockSpec((blk, 1), lambda i: (i, 0)),
        ],
        out_shape=[
            jax.ShapeDtypeStruct((NPAD, HID), _f32),
            jax.ShapeDtypeStruct((NPAD, 1), _f32),
            jax.ShapeDtypeStruct((NPAD, 1), _f32),
        ],
    )(p_part, d0, d1, bias, Wg_l, ats, atd)


def _kdec_body(p_ref, d0_ref, d1_ref, bias_ref, wdec_ref, bdec_ref, out_ref):
    h = _finish(p_ref, d0_ref, d1_ref, bias_ref)
    out_ref[...] = jnp.dot(h, wdec_ref[...], **_DOT) + bdec_ref[...][None, :]


def _kdec(p_part, d0, d1, bias, W_dec, b_dec):
    blk = 128
    return pl.pallas_call(
        _kdec_body,
        grid=(NPAD // blk,),
        in_specs=[
            pl.BlockSpec((NC, blk, HID), lambda i: (0, i, 0)),
            pl.BlockSpec((blk, 1), lambda i: (i, 0)),
            pl.BlockSpec((blk, 1), lambda i: (i, 0)),
            pl.BlockSpec((HID,), lambda i: (0,)),
            pl.BlockSpec((HID, OUT_F * FH), lambda i: (0, 0)),
            pl.BlockSpec((OUT_F * FH,), lambda i: (0,)),
        ],
        out_specs=pl.BlockSpec((blk, OUT_F * FH), lambda i: (i, 0)),
        out_shape=jax.ShapeDtypeStruct((NPAD, OUT_F * FH), _f32),
    )(p_part, d0, d1, bias, W_dec, b_dec)


def kernel(x, edge_index, edge_attr, W_emb, b_emb, ln_g, ln_b,
           Wg, att_s, att_d, We, att_e, bg, W_dec, b_dec):
    src = edge_index[0]
    dst = edge_index[1]
    npe = EPAD - E
    pad_idx = (jnp.arange(npe, dtype=_i32) % (NPAD - N)) + N
    src_p = jnp.concatenate([src, pad_idx])
    dst_p = jnp.concatenate([dst, pad_idx])
    ea_p = jnp.concatenate([edge_attr, jnp.zeros((npe, ED), _f32)], axis=0)
    x2 = jnp.pad(x.reshape(N, IN_F * INPUT_SIZE), ((0, NPAD - N), (0, 0)))

    z_acc = jnp.zeros((SLC, HID), _f32)
    z_den = jnp.zeros((SLC,), _f32)
    z_ea = jnp.zeros((SLC, ED), _f32)
    qneg = jnp.full((NPAD,), QNEG, _i32)

    deg_part, easum_part = _build_p1()(dst_p, ea_p, z_ea, z_den)
    AE = _aek(ea_p, We, att_e)
    LAE = _laek(easum_part, deg_part, We, att_e)

    H, a_s, a_d = _k0(x2, W_emb, b_emb, ln_g, ln_b,
                      Wg[0], att_s[0][:, None], att_d[0][:, None])
    out = None
    for l in range(L5):
        asrc = a_s[:, 0]
        adst = a_d[:, 0]
        ee, el, qmax_part = _build_sca()(
            src_p, dst_p, AE[l], LAE[l], asrc, adst, qneg)
        p_part, den_part = _build_scbc()(
            src_p, dst_p, ee, el, qmax_part, H, z_acc, z_den)
        d0 = den_part[0][:, None]
        d1 = den_part[1][:, None]
        if l < L5 - 1:
            H, a_s, a_d = _kl(p_part, d0, d1, bg[l],
                              Wg[l + 1], att_s[l + 1][:, None],
                              att_d[l + 1][:, None])
        else:
            out = _kdec(p_part, d0, d1, bg[l], W_dec, b_dec)
    return out[:N].reshape(N, OUT_F, FH)


# speedup vs baseline: 14.3496x; 1.0715x over previous
import functools

import jax
import jax.numpy as jnp
from jax import lax
from jax.experimental import pallas as pl
from jax.experimental.pallas import tpu as pltpu
from jax.experimental.pallas import tpu_sc as plsc

N = 10000
IN_F = 32
INPUT_SIZE = 4
OUT_F = 32
FH = 4
HID = 128
ED = 16
L5 = 5
E = 320000

NC = 2
NS = 16
NW = NC * NS
NPAD = 10240
EPAD = 327680
CHUNK = 128
EW = EPAD // NW
NCHUNKS = EW // CHUNK
SLC = NPAD // NS
WN = NPAD // NW
QNEG = -(1 << 30)
QCLIP = 100000.0

_f32 = jnp.float32
_i32 = jnp.int32
_u32 = jnp.uint32


def _mesh():
    return plsc.VectorSubcoreMesh(core_axis_name="c", subcore_axis_name="s")


def _wid():
    c = lax.axis_index("c")
    s = lax.axis_index("s")
    return c, s, s * NC + c


@functools.lru_cache(maxsize=None)
def _build_p1():
    @functools.partial(
        pl.kernel,
        mesh=_mesh(),
        compiler_params=pltpu.CompilerParams(needs_layout_passes=False),
        out_type=[
            jax.ShapeDtypeStruct((NC, NPAD), _f32),
            jax.ShapeDtypeStruct((NC, NPAD, ED), _f32),
        ],
        scratch_types=[
            pltpu.VMEM((1, CHUNK), _i32),
            pltpu.VMEM((CHUNK, ED), _f32),
            pltpu.VMEM((1, CHUNK), _f32),
            pltpu.VMEM_SHARED((NPAD,), _f32),
            pltpu.VMEM_SHARED((NPAD, ED), _f32),
        ],
    )
    def p1(dst_hbm, ea_hbm, zea_hbm, zden_hbm, deg_out, easum_out,
           idxv, eav, ones, deg_sh, ea_sh):
        c, s, w = _wid()
        pltpu.sync_copy(zden_hbm, deg_sh.at[pl.ds(s * SLC, SLC)])
        pltpu.sync_copy(zea_hbm, ea_sh.at[pl.ds(s * SLC, SLC), :])
        for g in range(CHUNK // 16):
            ones[0, pl.ds(g * 16, 16)] = jnp.full((16,), 1.0, _f32)
        plsc.subcore_barrier()

        def body(i, carry):
            base = w * EW + i * CHUNK
            pltpu.sync_copy(dst_hbm.at[pl.ds(base, CHUNK)], idxv.at[0])
            pltpu.sync_copy(ea_hbm.at[pl.ds(base, CHUNK), :], eav)
            pltpu.sync_copy(eav, ea_sh.at[idxv.at[0]], add=True)
            pltpu.sync_copy(ones.at[0], deg_sh.at[idxv.at[0]], add=True)
            return carry

        lax.fori_loop(0, NCHUNKS, body, 0)
        plsc.subcore_barrier()
        pltpu.sync_copy(deg_sh.at[pl.ds(s * SLC, SLC)],
                        deg_out.at[c, pl.ds(s * SLC, SLC)])
        pltpu.sync_copy(ea_sh.at[pl.ds(s * SLC, SLC), :],
                        easum_out.at[c, pl.ds(s * SLC, SLC), :])

    return p1


@functools.lru_cache(maxsize=None)
def _build_sca():
    @functools.partial(
        pl.kernel,
        mesh=_mesh(),
        compiler_params=pltpu.CompilerParams(needs_layout_passes=False),
        out_type=[
            jax.ShapeDtypeStruct((EPAD,), _f32),
            jax.ShapeDtypeStruct((NPAD,), _f32),
            jax.ShapeDtypeStruct((NC, NPAD), _i32),
        ],
        scratch_types=[
            pltpu.VMEM((NPAD,), _f32),
            pltpu.VMEM((NPAD,), _f32),
            pltpu.VMEM((NPAD,), _i32),
            pltpu.VMEM((2, 2, CHUNK), _i32),
            pltpu.VMEM((2, CHUNK), _f32),
            pltpu.VMEM((CHUNK,), _f32),
            pltpu.VMEM((32,), _i32),
            pltpu.VMEM((WN,), _f32),
            pltpu.VMEM((WN,), _f32),
            pltpu.VMEM((SLC,), _i32),
            pltpu.VMEM((SLC,), _i32),
            pltpu.SemaphoreType.DMA((2,)),
            pltpu.VMEM_SHARED((NS, NPAD), _i32),
        ],
    )
    def sca(sd_hbm, ae_hbm, lae_hbm, asrc_hbm, adst_hbm, qneg_hbm,
            ee_out, el_out, qmax_out,
            asrc_t, adst_t, qmax_t, sdv, aev, ebuf, scr, laev, elv,
            racc, rtmp, isem, qsh):
        c, s, w = _wid()
        pltpu.sync_copy(asrc_hbm, asrc_t)
        pltpu.sync_copy(adst_hbm, adst_t)
        pltpu.sync_copy(qneg_hbm, qmax_t)
        scr[pl.ds(16, 16)] = jnp.full((16,), -1, _i32)
        iota1 = lax.iota(_i32, 16) + 1

        jc0 = w * NCHUNKS
        pltpu.async_copy(sd_hbm.at[jc0], sdv.at[0], isem.at[0])
        pltpu.async_copy(ae_hbm.at[pl.ds(jc0 * CHUNK, CHUNK)], aev.at[0],
                         isem.at[0])

        def wait_in(b):
            pltpu.make_async_copy(sd_hbm.at[jc0], sdv.at[b],
                                  isem.at[b]).wait()
            pltpu.make_async_copy(ae_hbm.at[pl.ds(0, CHUNK)], aev.at[b],
                                  isem.at[b]).wait()

        def body(i, carry):
            cur = lax.rem(i, 2)
            nxt = 1 - cur

            @pl.when(i < NCHUNKS - 1)
            def _():
                base = (jc0 + i + 1) * CHUNK
                pltpu.async_copy(sd_hbm.at[jc0 + i + 1], sdv.at[nxt],
                                 isem.at[nxt])
                pltpu.async_copy(ae_hbm.at[pl.ds(base, CHUNK)], aev.at[nxt],
                                 isem.at[nxt])

            wait_in(cur)
            base = (jc0 + i) * CHUNK
            sdc = sdv.at[cur]
            aec = aev.at[cur]
            for g in range(CHUNK // 16):
                sl = pl.ds(g * 16, 16)
                s16 = sdc[0, sl]
                d16 = sdc[1, sl]
                a1 = plsc.load_gather(asrc_t, [s16])
                a2 = plsc.load_gather(adst_t, [d16])
                e = a1 + a2 + aec[sl]
                e = jnp.where(e >= 0.0, e, e * 0.2)
                ebuf[sl] = e
                q = jnp.clip(e, -QCLIP, QCLIP).astype(_i32)
                ku = (lax.bitcast_convert_type(d16, _u32) << 18) | \
                     lax.bitcast_convert_type(q + 131072, _u32)
                ks, _unused = plsc.sort_key_val(ku, ku)
                scr[pl.ds(0, 16)] = lax.bitcast_convert_type(ks, _i32)
                nxt = lax.bitcast_convert_type(
                    plsc.load_gather(scr, [iota1]), _u32)
                mask = (ks >> 18) != (nxt >> 18)
                dsort = lax.bitcast_convert_type(ks >> 18, _i32)
                qsort = lax.bitcast_convert_type(
                    ks & jnp.uint32(0x3FFFF), _i32) - 131072
                cur = plsc.load_gather(qmax_t, [dsort])
                plsc.store_scatter(qmax_t, [dsort],
                                   jnp.maximum(cur, qsort), mask=mask)
            pltpu.sync_copy(ebuf, ee_out.at[pl.ds(base, CHUNK)])
            return carry

        lax.fori_loop(0, NCHUNKS, body, 0)

        n0 = w * WN
        pltpu.sync_copy(lae_hbm.at[pl.ds(n0, WN)], laev)
        for g in range(WN // 16):
            sl16 = pl.ds(n0 + g * 16, 16)
            sl = pl.ds(g * 16, 16)
            e = asrc_t[sl16] + adst_t[sl16] + laev[sl]
            e = jnp.where(e >= 0.0, e, e * 0.2)
            elv[sl] = e
            q = jnp.clip(e, -QCLIP, QCLIP).astype(_i32)
            qmax_t[sl16] = jnp.maximum(qmax_t[sl16], q)
        pltpu.sync_copy(elv, el_out.at[pl.ds(n0, WN)])

        pltpu.sync_copy(qmax_t, qsh.at[s])
        plsc.subcore_barrier()
        pltpu.sync_copy(qsh.at[0, pl.ds(s * SLC, SLC)], racc)
        for r in range(1, NS):
            pltpu.sync_copy(qsh.at[r, pl.ds(s * SLC, SLC)], rtmp)
            for g in range(SLC // 16):
                sl = pl.ds(g * 16, 16)
                racc[sl] = jnp.maximum(racc[sl], rtmp[sl])
        pltpu.sync_copy(racc, qmax_out.at[c, pl.ds(s * SLC, SLC)])

    return sca


@functools.lru_cache(maxsize=None)
def _build_scbc():
    @functools.partial(
        pl.kernel,
        mesh=_mesh(),
        compiler_params=pltpu.CompilerParams(needs_layout_passes=False),
        out_type=[
            jax.ShapeDtypeStruct((NC, NPAD, HID), _f32),
            jax.ShapeDtypeStruct((NC, NPAD), _f32),
        ],
        scratch_types=[
            pltpu.VMEM((NPAD,), _i32),
            pltpu.VMEM((2, 2, CHUNK), _i32),
            pltpu.VMEM((1, CHUNK), _i32),
            pltpu.VMEM((2, CHUNK), _f32),
            pltpu.VMEM((2, CHUNK), _f32),
            pltpu.VMEM((2, CHUNK, HID), _f32),
            pltpu.SemaphoreType.DMA((2,)),
            pltpu.SemaphoreType.DMA((2,)),
            pltpu.VMEM_SHARED((NPAD, HID), _f32),
            pltpu.VMEM_SHARED((NPAD,), _f32),
        ],
    )
    def scbc(sd_hbm, ee_hbm, el_hbm, qmax_hbm, h_hbm,
             zacc_hbm, zden_hbm,
             p_out, den_out,
             bq_t, sdv, linv, ev, exv, rows, gsem, ssem,
             acc_sh, den_sh):
        c, s, w = _wid()
        pltpu.sync_copy(qmax_hbm, bq_t)
        pltpu.sync_copy(zacc_hbm, acc_sh.at[pl.ds(s * SLC, SLC), :])
        pltpu.sync_copy(zden_hbm, den_sh.at[pl.ds(s * SLC, SLC)])
        plsc.subcore_barrier()

        def scale_rows(cur):
            rowc = rows.at[cur]
            exc = exv.at[cur]

            def gbody(g, carry):
                ex16 = exc[pl.ds(g * 16, 16)]
                base = g * 16
                for rr in range(16):
                    xr = ex16[rr]
                    r = base + rr
                    for k in range(HID // 16):
                        sl = pl.ds(k * 16, 16)
                        rowc[r, sl] = rowc[r, sl] * xr
                return carry
            lax.fori_loop(0, CHUNK // 16, gbody, 0)

        for k in range(3):
            j = k * NW + w

            @pl.when(j < NPAD // CHUNK)
            def _():
                i0 = j * CHUNK
                pltpu.sync_copy(el_hbm.at[pl.ds(i0, CHUNK)], ev.at[0])
                pltpu.sync_copy(h_hbm.at[pl.ds(i0, CHUNK), :], rows.at[0])
                for g in range(CHUNK // 16):
                    sl = pl.ds(g * 16, 16)
                    b16 = bq_t[pl.ds(i0 + g * 16, 16)].astype(_f32)
                    exv.at[0][sl] = jnp.exp(ev.at[0][sl] - b16)
                    linv[0, sl] = lax.iota(_i32, 16) + (i0 + g * 16)
                scale_rows(0)
                pltpu.sync_copy(rows.at[0], acc_sh.at[linv.at[0]], add=True)
                pltpu.sync_copy(exv.at[0], den_sh.at[linv.at[0]], add=True)

        jc0 = w * NCHUNKS
        pltpu.sync_copy(sd_hbm.at[jc0], sdv.at[0])
        pltpu.sync_copy(ee_hbm.at[pl.ds(jc0 * CHUNK, CHUNK)], ev.at[0])
        pltpu.async_copy(h_hbm.at[sdv.at[0, 0]], rows.at[0], gsem.at[0])

        def drain_scatter(b):
            pltpu.make_async_copy(
                h_hbm.at[pl.ds(0, CHUNK), :], rows.at[b], ssem.at[b]).wait()
            pltpu.make_async_copy(
                ee_hbm.at[pl.ds(0, CHUNK)], exv.at[b], ssem.at[b]).wait()

        def body(i, carry):
            cur = lax.rem(i, 2)
            nxt = 1 - cur

            @pl.when(i < NCHUNKS - 1)
            def _():
                base = (jc0 + i + 1) * CHUNK
                pltpu.sync_copy(sd_hbm.at[jc0 + i + 1], sdv.at[nxt])
                pltpu.sync_copy(ee_hbm.at[pl.ds(base, CHUNK)], ev.at[nxt])

                @pl.when(i > 0)
                def _():
                    drain_scatter(nxt)
                pltpu.async_copy(h_hbm.at[sdv.at[nxt, 0]], rows.at[nxt],
                                 gsem.at[nxt])

            pltpu.make_async_copy(
                h_hbm.at[pl.ds(0, CHUNK), :], rows.at[cur],
                gsem.at[cur]).wait()
            sdc = sdv.at[cur]
            evc = ev.at[cur]
            exc = exv.at[cur]
            for g in range(CHUNK // 16):
                sl = pl.ds(g * 16, 16)
                d16 = sdc[1, sl]
                b16 = plsc.load_gather(bq_t, [d16]).astype(_f32)
                exc[sl] = jnp.exp(evc[sl] - b16)
            scale_rows(cur)
            pltpu.async_copy(rows.at[cur], acc_sh.at[sdv.at[cur, 1]],
                             ssem.at[cur], add=True)
            pltpu.async_copy(exv.at[cur], den_sh.at[sdv.at[cur, 1]],
                             ssem.at[cur], add=True)
            return carry

        lax.fori_loop(0, NCHUNKS, body, 0)
        drain_scatter(0)
        drain_scatter(1)
        plsc.subcore_barrier()
        pltpu.sync_copy(acc_sh.at[pl.ds(s * SLC, SLC), :],
                        p_out.at[c, pl.ds(s * SLC, SLC), :])
        pltpu.sync_copy(den_sh.at[pl.ds(s * SLC, SLC)],
                        den_out.at[c, pl.ds(s * SLC, SLC)])

    return scbc


_DOT = dict(precision=lax.Precision.HIGHEST, preferred_element_type=_f32)


def _wae8(we, ate):
    wae = lax.dot_general(we, ate, (((2,), (1,)), ((0,), (0,))), **_DOT)
    return jnp.concatenate([wae, jnp.zeros((8 - L5, ED), _f32)], axis=0)


def _waek_body(we_ref, ate_ref, out_ref):
    out_ref[...] = _wae8(we_ref[...], ate_ref[...])


def _waek(We, att_e):
    return pl.pallas_call(
        _waek_body,
        out_shape=jax.ShapeDtypeStruct((8, ED), _f32),
    )(We, att_e)


def _qmergek_body(q_ref, out_ref):
    out_ref[...] = jnp.maximum(q_ref[0], q_ref[1])


def _qmergek(qmax_part):
    return pl.pallas_call(
        _qmergek_body,
        out_shape=jax.ShapeDtypeStruct((NPAD,), _i32),
    )(qmax_part)


def _aek_body(ea_ref, wae_ref, out_ref):
    out_ref[...] = lax.dot_general(wae_ref[...], ea_ref[...],
                                   (((1,), (1,)), ((), ())), **_DOT)


def _aek(ea_p, wae):
    be = 8192
    return pl.pallas_call(
        _aek_body,
        grid=(EPAD // be,),
        in_specs=[
            pl.BlockSpec((be, ED), lambda i: (i, 0)),
            pl.BlockSpec((8, ED), lambda i: (0, 0)),
        ],
        out_specs=pl.BlockSpec((8, be), lambda i: (0, i)),
        out_shape=jax.ShapeDtypeStruct((8, EPAD), _f32),
    )(ea_p, wae)


def _laek_body(easum_ref, deg_ref, wae_ref, out_ref):
    ea = easum_ref[0] + easum_ref[1]
    lae = lax.dot_general(wae_ref[...], ea, (((1,), (1,)), ((), ())), **_DOT)
    deg = jnp.maximum(deg_ref[0] + deg_ref[1], 1.0)[None, :]
    out_ref[...] = lae / deg


def _laek(easum_part, deg_part, wae):
    blk = 1024
    return pl.pallas_call(
        _laek_body,
        grid=(NPAD // blk,),
        in_specs=[
            pl.BlockSpec((NC, blk, ED), lambda i: (0, i, 0)),
            pl.BlockSpec((NC, blk), lambda i: (0, i)),
            pl.BlockSpec((8, ED), lambda i: (0, 0)),
        ],
        out_specs=pl.BlockSpec((8, blk), lambda i: (0, i)),
        out_shape=jax.ShapeDtypeStruct((8, NPAD), _f32),
    )(easum_part, deg_part, wae)


def _head_tail(h, wg_ref, ats_ref, atd_ref, h_ref, as_ref, ad_ref):
    hn = jnp.dot(h, wg_ref[...], **_DOT)
    h_ref[...] = hn
    as_ref[...] = jnp.dot(hn, ats_ref[...], **_DOT)
    ad_ref[...] = jnp.dot(hn, atd_ref[...], **_DOT)


def _k0_body(x_ref, wemb_ref, bemb_ref, lng_ref, lnb_ref,
             wg_ref, ats_ref, atd_ref, h_ref, as_ref, ad_ref):
    h = jnp.dot(x_ref[...], wemb_ref[...], **_DOT) + bemb_ref[...][None, :]
    m = jnp.mean(h, axis=-1, keepdims=True)
    v = jnp.mean((h - m) ** 2, axis=-1, keepdims=True)
    h = (h - m) / jnp.sqrt(v + 1e-5) * lng_ref[...][None, :] \
        + lnb_ref[...][None, :]
    h = jnp.maximum(h, 0.0)
    _head_tail(h, wg_ref, ats_ref, atd_ref, h_ref, as_ref, ad_ref)


def _k0(x2, W_emb, b_emb, ln_g, ln_b, Wg0, ats0, atd0):
    blk = 512
    return pl.pallas_call(
        _k0_body,
        grid=(NPAD // blk,),
        in_specs=[
            pl.BlockSpec((blk, HID), lambda i: (i, 0)),
            pl.BlockSpec((HID, HID), lambda i: (0, 0)),
            pl.BlockSpec((HID,), lambda i: (0,)),
            pl.BlockSpec((HID,), lambda i: (0,)),
            pl.BlockSpec((HID,), lambda i: (0,)),
            pl.BlockSpec((HID, HID), lambda i: (0, 0)),
            pl.BlockSpec((HID, 1), lambda i: (0, 0)),
            pl.BlockSpec((HID, 1), lambda i: (0, 0)),
        ],
        out_specs=[
            pl.BlockSpec((blk, HID), lambda i: (i, 0)),
            pl.BlockSpec((blk, 1), lambda i: (i, 0)),
            pl.BlockSpec((blk, 1), lambda i: (i, 0)),
        ],
        out_shape=[
            jax.ShapeDtypeStruct((NPAD, HID), _f32),
            jax.ShapeDtypeStruct((NPAD, 1), _f32),
            jax.ShapeDtypeStruct((NPAD, 1), _f32),
        ],
    )(x2, W_emb, b_emb, ln_g, ln_b, Wg0, ats0, atd0)


def _finish(p_ref, d0_ref, d1_ref, bias_ref):
    p = p_ref[0] + p_ref[1]
    den = d0_ref[...] + d1_ref[...] + 1e-16
    return jnp.maximum(p / den + bias_ref[...][None, :], 0.0)


def _kl_body(p_ref, d0_ref, d1_ref, bias_ref,
             wg_ref, ats_ref, atd_ref, h_ref, as_ref, ad_ref):
    h = _finish(p_ref, d0_ref, d1_ref, bias_ref)
    _head_tail(h, wg_ref, ats_ref, atd_ref, h_ref, as_ref, ad_ref)


def _kl(p_part, d0, d1, bias, Wg_l, ats, atd):
    blk = 512
    return pl.pallas_call(
        _kl_body,
        grid=(NPAD // blk,),
        in_specs=[
            pl.BlockSpec((NC, blk, HID), lambda i: (0, i, 0)),
            pl.BlockSpec((blk, 1), lambda i: (i, 0)),
            pl.BlockSpec((blk, 1), lambda i: (i, 0)),
            pl.BlockSpec((HID,), lambda i: (0,)),
            pl.BlockSpec((HID, HID), lambda i: (0, 0)),
            pl.BlockSpec((HID, 1), lambda i: (0, 0)),
            pl.BlockSpec((HID, 1), lambda i: (0, 0)),
        ],
        out_specs=[
            pl.BlockSpec((blk, HID), lambda i: (i, 0)),
            pl.BlockSpec((blk, 1), lambda i: (i, 0)),
            pl.BlockSpec((blk, 1), lambda i: (i, 0)),
        ],
        out_shape=[
            jax.ShapeDtypeStruct((NPAD, HID), _f32),
            jax.ShapeDtypeStruct((NPAD, 1), _f32),
            jax.ShapeDtypeStruct((NPAD, 1), _f32),
        ],
    )(p_part, d0, d1, bias, Wg_l, ats, atd)


def _kdec_body(p_ref, d0_ref, d1_ref, bias_ref, wdec_ref, bdec_ref, out_ref):
    h = _finish(p_ref, d0_ref, d1_ref, bias_ref)
    out_ref[...] = jnp.dot(h, wdec_ref[...], **_DOT) + bdec_ref[...][None, :]


def _kdec(p_part, d0, d1, bias, W_dec, b_dec):
    blk = 512
    return pl.pallas_call(
        _kdec_body,
        grid=(NPAD // blk,),
        in_specs=[
            pl.BlockSpec((NC, blk, HID), lambda i: (0, i, 0)),
            pl.BlockSpec((blk, 1), lambda i: (i, 0)),
            pl.BlockSpec((blk, 1), lambda i: (i, 0)),
            pl.BlockSpec((HID,), lambda i: (0,)),
            pl.BlockSpec((HID, OUT_F * FH), lambda i: (0, 0)),
            pl.BlockSpec((OUT_F * FH,), lambda i: (0,)),
        ],
        out_specs=pl.BlockSpec((blk, OUT_F * FH), lambda i: (i, 0)),
        out_shape=jax.ShapeDtypeStruct((NPAD, OUT_F * FH), _f32),
    )(p_part, d0, d1, bias, W_dec, b_dec)


def kernel(x, edge_index, edge_attr, W_emb, b_emb, ln_g, ln_b,
           Wg, att_s, att_d, We, att_e, bg, W_dec, b_dec):
    src = edge_index[0]
    dst = edge_index[1]
    npe = EPAD - E
    pad_idx = (jnp.arange(npe, dtype=_i32) % (NPAD - N)) + N
    src_p = jnp.concatenate([src, pad_idx])
    dst_p = jnp.concatenate([dst, pad_idx])
    ea_p = jnp.concatenate([edge_attr, jnp.zeros((npe, ED), _f32)], axis=0)
    x2 = jnp.pad(x.reshape(N, IN_F * INPUT_SIZE), ((0, NPAD - N), (0, 0)))

    z_acc = jnp.zeros((SLC, HID), _f32)
    z_den = jnp.zeros((SLC,), _f32)
    z_ea = jnp.zeros((SLC, ED), _f32)
    qneg = jnp.full((NPAD,), QNEG, _i32)

    sd = jnp.stack([src_p.reshape(EPAD // CHUNK, CHUNK),
                    dst_p.reshape(EPAD // CHUNK, CHUNK)], axis=1)
    deg_part, easum_part = _build_p1()(dst_p, ea_p, z_ea, z_den)
    wae = _waek(We, att_e)
    AE = _aek(ea_p, wae)
    LAE = _laek(easum_part, deg_part, wae)

    H, a_s, a_d = _k0(x2, W_emb, b_emb, ln_g, ln_b,
                      Wg[0], att_s[0][:, None], att_d[0][:, None])
    out = None
    for l in range(L5):
        asrc = a_s[:, 0]
        adst = a_d[:, 0]
        ee, el, qmax_part = _build_sca()(
            sd, AE[l], LAE[l], asrc, adst, qneg)
        qmax = _qmergek(qmax_part)
        p_part, den_part = _build_scbc()(
            sd, ee, el, qmax, H, z_acc, z_den)
        d0 = den_part[0][:, None]
        d1 = den_part[1][:, None]
        if l < L5 - 1:
            H, a_s, a_d = _kl(p_part, d0, d1, bg[l],
                              Wg[l + 1], att_s[l + 1][:, None],
                              att_d[l + 1][:, None])
        else:
            out = _kdec(p_part, d0, d1, bg[l], W_dec, b_dec)
    return out[:N].reshape(N, OUT_F, FH)


# speedup vs baseline: 23.9590x; 1.6697x over previous
import functools

import jax
import jax.numpy as jnp
from jax import lax
from jax.experimental import pallas as pl
from jax.experimental.pallas import tpu as pltpu
from jax.experimental.pallas import tpu_sc as plsc

N = 10000
IN_F = 32
INPUT_SIZE = 4
OUT_F = 32
FH = 4
HID = 128
ED = 16
L5 = 5
E = 320000

NC = 2
NS = 16
NW = NC * NS
NPAD = 10240
EPAD = 327680
CHUNK = 128
EW = EPAD // NW
NCHUNKS = EW // CHUNK
SLC = NPAD // NS
WN = NPAD // NW
QNEG = -(1 << 30)
QCLIP = 100000.0

_f32 = jnp.float32
_i32 = jnp.int32
_u32 = jnp.uint32


def _mesh():
    return plsc.VectorSubcoreMesh(core_axis_name="c", subcore_axis_name="s")


def _wid():
    c = lax.axis_index("c")
    s = lax.axis_index("s")
    return c, s, s * NC + c


@functools.lru_cache(maxsize=None)
def _build_p1():
    @functools.partial(
        pl.kernel,
        mesh=_mesh(),
        compiler_params=pltpu.CompilerParams(needs_layout_passes=False),
        out_type=[
            jax.ShapeDtypeStruct((NC, NPAD), _f32),
            jax.ShapeDtypeStruct((NC, NPAD, ED), _f32),
        ],
        scratch_types=[
            pltpu.VMEM((1, CHUNK), _i32),
            pltpu.VMEM((CHUNK, ED), _f32),
            pltpu.VMEM((1, CHUNK), _f32),
            pltpu.VMEM_SHARED((NPAD,), _f32),
            pltpu.VMEM_SHARED((NPAD, ED), _f32),
        ],
    )
    def p1(dst_hbm, ea_hbm, zea_hbm, zden_hbm, deg_out, easum_out,
           idxv, eav, ones, deg_sh, ea_sh):
        c, s, w = _wid()
        pltpu.sync_copy(zden_hbm, deg_sh.at[pl.ds(s * SLC, SLC)])
        pltpu.sync_copy(zea_hbm, ea_sh.at[pl.ds(s * SLC, SLC), :])
        for g in range(CHUNK // 16):
            ones[0, pl.ds(g * 16, 16)] = jnp.full((16,), 1.0, _f32)
        plsc.subcore_barrier()

        def body(i, carry):
            base = w * EW + i * CHUNK
            pltpu.sync_copy(dst_hbm.at[pl.ds(base, CHUNK)], idxv.at[0])
            pltpu.sync_copy(ea_hbm.at[pl.ds(base, CHUNK), :], eav)
            pltpu.sync_copy(eav, ea_sh.at[idxv.at[0]], add=True)
            pltpu.sync_copy(ones.at[0], deg_sh.at[idxv.at[0]], add=True)
            return carry

        lax.fori_loop(0, NCHUNKS, body, 0)
        plsc.subcore_barrier()
        pltpu.sync_copy(deg_sh.at[pl.ds(s * SLC, SLC)],
                        deg_out.at[c, pl.ds(s * SLC, SLC)])
        pltpu.sync_copy(ea_sh.at[pl.ds(s * SLC, SLC), :],
                        easum_out.at[c, pl.ds(s * SLC, SLC), :])

    return p1


@functools.lru_cache(maxsize=None)
def _build_sca():
    @functools.partial(
        pl.kernel,
        mesh=_mesh(),
        compiler_params=pltpu.CompilerParams(needs_layout_passes=False),
        out_type=[
            jax.ShapeDtypeStruct((EPAD,), _f32),
            jax.ShapeDtypeStruct((NPAD,), _f32),
            jax.ShapeDtypeStruct((NC, NPAD), _i32),
        ],
        scratch_types=[
            pltpu.VMEM((NPAD,), _f32),
            pltpu.VMEM((NPAD,), _f32),
            pltpu.VMEM((NPAD,), _i32),
            pltpu.VMEM((2, 2, CHUNK), _i32),
            pltpu.VMEM((2, CHUNK), _f32),
            pltpu.VMEM((CHUNK,), _f32),
            pltpu.VMEM((32,), _i32),
            pltpu.VMEM((WN,), _f32),
            pltpu.VMEM((WN,), _f32),
            pltpu.VMEM((SLC,), _i32),
            pltpu.VMEM((SLC,), _i32),
            pltpu.SemaphoreType.DMA((2,)),
            pltpu.VMEM_SHARED((NS, NPAD), _i32),
        ],
    )
    def sca(sd_hbm, ae_hbm, lae_hbm, asrc_hbm, adst_hbm, qneg_hbm,
            ee_out, el_out, qmax_out,
            asrc_t, adst_t, qmax_t, sdv, aev, ebuf, scr, laev, elv,
            racc, rtmp, isem, qsh):
        c, s, w = _wid()
        pltpu.sync_copy(asrc_hbm, asrc_t)
        pltpu.sync_copy(adst_hbm, adst_t)
        pltpu.sync_copy(qneg_hbm, qmax_t)
        scr[pl.ds(16, 16)] = jnp.full((16,), -1, _i32)
        iota1 = lax.iota(_i32, 16) + 1

        jc0 = w * NCHUNKS
        pltpu.async_copy(sd_hbm.at[jc0], sdv.at[0], isem.at[0])
        pltpu.async_copy(ae_hbm.at[pl.ds(jc0 * CHUNK, CHUNK)], aev.at[0],
                         isem.at[0])

        def wait_in(b):
            pltpu.make_async_copy(sd_hbm.at[jc0], sdv.at[b],
                                  isem.at[b]).wait()
            pltpu.make_async_copy(ae_hbm.at[pl.ds(0, CHUNK)], aev.at[b],
                                  isem.at[b]).wait()

        def process(i, b):
            nb = 1 - b

            @pl.when(i < NCHUNKS - 1)
            def _():
                base = (jc0 + i + 1) * CHUNK
                pltpu.async_copy(sd_hbm.at[jc0 + i + 1], sdv.at[nb],
                                 isem.at[nb])
                pltpu.async_copy(ae_hbm.at[pl.ds(base, CHUNK)], aev.at[nb],
                                 isem.at[nb])

            wait_in(b)
            base = (jc0 + i) * CHUNK
            sdc = sdv.at[b]
            aec = aev.at[b]
            for g in range(CHUNK // 16):
                sl = pl.ds(g * 16, 16)
                s16 = sdc[0, sl]
                d16 = sdc[1, sl]
                a1 = plsc.load_gather(asrc_t, [s16])
                a2 = plsc.load_gather(adst_t, [d16])
                e = a1 + a2 + aec[sl]
                e = jnp.where(e >= 0.0, e, e * 0.2)
                ebuf[sl] = e
                q = jnp.clip(e, -QCLIP, QCLIP).astype(_i32)
                ku = (lax.bitcast_convert_type(d16, _u32) << 18) | \
                     lax.bitcast_convert_type(q + 131072, _u32)
                ks, _unused = plsc.sort_key_val(ku, ku)
                scr[pl.ds(0, 16)] = lax.bitcast_convert_type(ks, _i32)
                nxt = lax.bitcast_convert_type(
                    plsc.load_gather(scr, [iota1]), _u32)
                mask = (ks >> 18) != (nxt >> 18)
                dsort = lax.bitcast_convert_type(ks >> 18, _i32)
                qsort = lax.bitcast_convert_type(
                    ks & jnp.uint32(0x3FFFF), _i32) - 131072
                cur = plsc.load_gather(qmax_t, [dsort])
                plsc.store_scatter(qmax_t, [dsort],
                                   jnp.maximum(cur, qsort), mask=mask)
            pltpu.sync_copy(ebuf, ee_out.at[pl.ds(base, CHUNK)])

        def body(ip, carry):
            process(ip * 2, 0)
            process(ip * 2 + 1, 1)
            return carry

        lax.fori_loop(0, NCHUNKS // 2, body, 0)

        n0 = w * WN
        pltpu.sync_copy(lae_hbm.at[pl.ds(n0, WN)], laev)
        for g in range(WN // 16):
            sl16 = pl.ds(n0 + g * 16, 16)
            sl = pl.ds(g * 16, 16)
            e = asrc_t[sl16] + adst_t[sl16] + laev[sl]
            e = jnp.where(e >= 0.0, e, e * 0.2)
            elv[sl] = e
            q = jnp.clip(e, -QCLIP, QCLIP).astype(_i32)
            qmax_t[sl16] = jnp.maximum(qmax_t[sl16], q)
        pltpu.sync_copy(elv, el_out.at[pl.ds(n0, WN)])

        pltpu.sync_copy(qmax_t, qsh.at[s])
        plsc.subcore_barrier()
        pltpu.sync_copy(qsh.at[0, pl.ds(s * SLC, SLC)], racc)
        for r in range(1, NS):
            pltpu.sync_copy(qsh.at[r, pl.ds(s * SLC, SLC)], rtmp)
            for g in range(SLC // 16):
                sl = pl.ds(g * 16, 16)
                racc[sl] = jnp.maximum(racc[sl], rtmp[sl])
        pltpu.sync_copy(racc, qmax_out.at[c, pl.ds(s * SLC, SLC)])

    return sca


@functools.lru_cache(maxsize=None)
def _build_scbc():
    @functools.partial(
        pl.kernel,
        mesh=_mesh(),
        compiler_params=pltpu.CompilerParams(needs_layout_passes=False),
        out_type=[
            jax.ShapeDtypeStruct((NC, NPAD, HID), _f32),
            jax.ShapeDtypeStruct((NC, NPAD), _f32),
        ],
        scratch_types=[
            pltpu.VMEM((NPAD,), _i32),
            pltpu.VMEM((2, 2, CHUNK), _i32),
            pltpu.VMEM((1, CHUNK), _i32),
            pltpu.VMEM((2, CHUNK), _f32),
            pltpu.VMEM((2, CHUNK), _f32),
            pltpu.VMEM((2, CHUNK, HID), _f32),
            pltpu.SemaphoreType.DMA((2,)),
            pltpu.SemaphoreType.DMA((2,)),
            pltpu.VMEM_SHARED((NPAD, HID), _f32),
            pltpu.VMEM_SHARED((NPAD,), _f32),
        ],
    )
    def scbc(sd_hbm, ee_hbm, el_hbm, qmax_hbm, h_hbm,
             zacc_hbm, zden_hbm,
             p_out, den_out,
             bq_t, sdv, linv, ev, exv, rows, gsem, ssem,
             acc_sh, den_sh):
        c, s, w = _wid()
        pltpu.sync_copy(qmax_hbm, bq_t)
        pltpu.sync_copy(zacc_hbm, acc_sh.at[pl.ds(s * SLC, SLC), :])
        pltpu.sync_copy(zden_hbm, den_sh.at[pl.ds(s * SLC, SLC)])
        plsc.subcore_barrier()

        def scale_rows(b):
            rowc = rows.at[b]
            exc = exv.at[b]

            def gbody(g, carry):
                ex16 = exc[pl.ds(g * 16, 16)]
                base = g * 16
                for rr in range(16):
                    xr = ex16[rr]
                    r = base + rr
                    for k in range(HID // 16):
                        sl = pl.ds(k * 16, 16)
                        rowc[r, sl] = rowc[r, sl] * xr
                return carry
            lax.fori_loop(0, CHUNK // 16, gbody, 0)

        for k in range(3):
            j = k * NW + w

            @pl.when(j < NPAD // CHUNK)
            def _():
                i0 = j * CHUNK
                pltpu.sync_copy(el_hbm.at[pl.ds(i0, CHUNK)], ev.at[0])
                pltpu.sync_copy(h_hbm.at[pl.ds(i0, CHUNK), :], rows.at[0])
                for g in range(CHUNK // 16):
                    sl = pl.ds(g * 16, 16)
                    b16 = bq_t[pl.ds(i0 + g * 16, 16)].astype(_f32)
                    exv.at[0][sl] = jnp.exp(ev.at[0][sl] - b16)
                    linv[0, sl] = lax.iota(_i32, 16) + (i0 + g * 16)
                scale_rows(0)
                pltpu.sync_copy(rows.at[0], acc_sh.at[linv.at[0]], add=True)
                pltpu.sync_copy(exv.at[0], den_sh.at[linv.at[0]], add=True)

        jc0 = w * NCHUNKS
        pltpu.sync_copy(sd_hbm.at[jc0], sdv.at[0])
        pltpu.sync_copy(ee_hbm.at[pl.ds(jc0 * CHUNK, CHUNK)], ev.at[0])
        pltpu.async_copy(h_hbm.at[sdv.at[0, 0]], rows.at[0], gsem.at[0])

        def drain_scatter(b):
            pltpu.make_async_copy(
                h_hbm.at[pl.ds(0, CHUNK), :], rows.at[b], ssem.at[b]).wait()
            pltpu.make_async_copy(
                ee_hbm.at[pl.ds(0, CHUNK)], exv.at[b], ssem.at[b]).wait()

        def process(i, b):
            nb = 1 - b

            @pl.when(i < NCHUNKS - 1)
            def _():
                @pl.when(i > 0)
                def _():
                    drain_scatter(nb)
                base = (jc0 + i + 1) * CHUNK
                pltpu.sync_copy(sd_hbm.at[jc0 + i + 1], sdv.at[nb])
                pltpu.sync_copy(ee_hbm.at[pl.ds(base, CHUNK)], ev.at[nb])
                pltpu.async_copy(h_hbm.at[sdv.at[nb, 0]], rows.at[nb],
                                 gsem.at[nb])

            pltpu.make_async_copy(
                h_hbm.at[pl.ds(0, CHUNK), :], rows.at[b],
                gsem.at[b]).wait()
            sdc = sdv.at[b]
            evc = ev.at[b]
            exc = exv.at[b]
            for g in range(CHUNK // 16):
                sl = pl.ds(g * 16, 16)
                d16 = sdc[1, sl]
                b16 = plsc.load_gather(bq_t, [d16]).astype(_f32)
                exc[sl] = jnp.exp(evc[sl] - b16)
            scale_rows(b)
            pltpu.async_copy(rows.at[b], acc_sh.at[sdv.at[b, 1]],
                             ssem.at[b], add=True)
            pltpu.async_copy(exv.at[b], den_sh.at[sdv.at[b, 1]],
                             ssem.at[b], add=True)

        def body(ip, carry):
            process(ip * 2, 0)
            process(ip * 2 + 1, 1)
            return carry

        lax.fori_loop(0, NCHUNKS // 2, body, 0)
        drain_scatter(0)
        drain_scatter(1)
        plsc.subcore_barrier()
        pltpu.sync_copy(acc_sh.at[pl.ds(s * SLC, SLC), :],
                        p_out.at[c, pl.ds(s * SLC, SLC), :])
        pltpu.sync_copy(den_sh.at[pl.ds(s * SLC, SLC)],
                        den_out.at[c, pl.ds(s * SLC, SLC)])

    return scbc


_DOT = dict(precision=lax.Precision.HIGHEST, preferred_element_type=_f32)


def _wae8(we, ate):
    wae = lax.dot_general(we, ate, (((2,), (1,)), ((0,), (0,))), **_DOT)
    return jnp.concatenate([wae, jnp.zeros((8 - L5, ED), _f32)], axis=0)


def _waek_body(we_ref, ate_ref, out_ref):
    out_ref[...] = _wae8(we_ref[...], ate_ref[...])


def _waek(We, att_e):
    return pl.pallas_call(
        _waek_body,
        out_shape=jax.ShapeDtypeStruct((8, ED), _f32),
    )(We, att_e)


def _qmergek_body(q_ref, out_ref):
    out_ref[...] = jnp.maximum(q_ref[0], q_ref[1])


def _qmergek(qmax_part):
    return pl.pallas_call(
        _qmergek_body,
        out_shape=jax.ShapeDtypeStruct((NPAD,), _i32),
    )(qmax_part)


def _aek_body(ea_ref, wae_ref, out_ref):
    out_ref[...] = lax.dot_general(wae_ref[...], ea_ref[...],
                                   (((1,), (1,)), ((), ())), **_DOT)


def _aek(ea_p, wae):
    be = 8192
    return pl.pallas_call(
        _aek_body,
        grid=(EPAD // be,),
        in_specs=[
            pl.BlockSpec((be, ED), lambda i: (i, 0)),
            pl.BlockSpec((8, ED), lambda i: (0, 0)),
        ],
        out_specs=pl.BlockSpec((8, be), lambda i: (0, i)),
        out_shape=jax.ShapeDtypeStruct((8, EPAD), _f32),
    )(ea_p, wae)


def _laek_body(easum_ref, deg_ref, wae_ref, out_ref):
    ea = easum_ref[0] + easum_ref[1]
    lae = lax.dot_general(wae_ref[...], ea, (((1,), (1,)), ((), ())), **_DOT)
    deg = jnp.maximum(deg_ref[0] + deg_ref[1], 1.0)[None, :]
    out_ref[...] = lae / deg


def _laek(easum_part, deg_part, wae):
    blk = 1024
    return pl.pallas_call(
        _laek_body,
        grid=(NPAD // blk,),
        in_specs=[
            pl.BlockSpec((NC, blk, ED), lambda i: (0, i, 0)),
            pl.BlockSpec((NC, blk), lambda i: (0, i)),
            pl.BlockSpec((8, ED), lambda i: (0, 0)),
        ],
        out_specs=pl.BlockSpec((8, blk), lambda i: (0, i)),
        out_shape=jax.ShapeDtypeStruct((8, NPAD), _f32),
    )(easum_part, deg_part, wae)


def _head_tail(h, wg_ref, ats_ref, atd_ref, h_ref, as_ref, ad_ref):
    hn = jnp.dot(h, wg_ref[...], **_DOT)
    h_ref[...] = hn
    as_ref[...] = jnp.dot(hn, ats_ref[...], **_DOT)
    ad_ref[...] = jnp.dot(hn, atd_ref[...], **_DOT)


def _k0_body(x_ref, wemb_ref, bemb_ref, lng_ref, lnb_ref,
             wg_ref, ats_ref, atd_ref, h_ref, as_ref, ad_ref):
    h = jnp.dot(x_ref[...], wemb_ref[...], **_DOT) + bemb_ref[...][None, :]
    m = jnp.mean(h, axis=-1, keepdims=True)
    v = jnp.mean((h - m) ** 2, axis=-1, keepdims=True)
    h = (h - m) / jnp.sqrt(v + 1e-5) * lng_ref[...][None, :] \
        + lnb_ref[...][None, :]
    h = jnp.maximum(h, 0.0)
    _head_tail(h, wg_ref, ats_ref, atd_ref, h_ref, as_ref, ad_ref)


def _k0(x2, W_emb, b_emb, ln_g, ln_b, Wg0, ats0, atd0):
    blk = 512
    return pl.pallas_call(
        _k0_body,
        grid=(NPAD // blk,),
        in_specs=[
            pl.BlockSpec((blk, HID), lambda i: (i, 0)),
            pl.BlockSpec((HID, HID), lambda i: (0, 0)),
            pl.BlockSpec((HID,), lambda i: (0,)),
            pl.BlockSpec((HID,), lambda i: (0,)),
            pl.BlockSpec((HID,), lambda i: (0,)),
            pl.BlockSpec((HID, HID), lambda i: (0, 0)),
            pl.BlockSpec((HID, 1), lambda i: (0, 0)),
            pl.BlockSpec((HID, 1), lambda i: (0, 0)),
        ],
        out_specs=[
            pl.BlockSpec((blk, HID), lambda i: (i, 0)),
            pl.BlockSpec((blk, 1), lambda i: (i, 0)),
            pl.BlockSpec((blk, 1), lambda i: (i, 0)),
        ],
        out_shape=[
            jax.ShapeDtypeStruct((NPAD, HID), _f32),
            jax.ShapeDtypeStruct((NPAD, 1), _f32),
            jax.ShapeDtypeStruct((NPAD, 1), _f32),
        ],
    )(x2, W_emb, b_emb, ln_g, ln_b, Wg0, ats0, atd0)


def _finish(p_ref, d0_ref, d1_ref, bias_ref):
    p = p_ref[0] + p_ref[1]
    den = d0_ref[...] + d1_ref[...] + 1e-16
    return jnp.maximum(p / den + bias_ref[...][None, :], 0.0)


def _kl_body(p_ref, d0_ref, d1_ref, bias_ref,
             wg_ref, ats_ref, atd_ref, h_ref, as_ref, ad_ref):
    h = _finish(p_ref, d0_ref, d1_ref, bias_ref)
    _head_tail(h, wg_ref, ats_ref, atd_ref, h_ref, as_ref, ad_ref)


def _kl(p_part, d0, d1, bias, Wg_l, ats, atd):
    blk = 512
    return pl.pallas_call(
        _kl_body,
        grid=(NPAD // blk,),
        in_specs=[
            pl.BlockSpec((NC, blk, HID), lambda i: (0, i, 0)),
            pl.BlockSpec((blk, 1), lambda i: (i, 0)),
            pl.BlockSpec((blk, 1), lambda i: (i, 0)),
            pl.BlockSpec((HID,), lambda i: (0,)),
            pl.BlockSpec((HID, HID), lambda i: (0, 0)),
            pl.BlockSpec((HID, 1), lambda i: (0, 0)),
            pl.BlockSpec((HID, 1), lambda i: (0, 0)),
        ],
        out_specs=[
            pl.BlockSpec((blk, HID), lambda i: (i, 0)),
            pl.BlockSpec((blk, 1), lambda i: (i, 0)),
            pl.BlockSpec((blk, 1), lambda i: (i, 0)),
        ],
        out_shape=[
            jax.ShapeDtypeStruct((NPAD, HID), _f32),
            jax.ShapeDtypeStruct((NPAD, 1), _f32),
            jax.ShapeDtypeStruct((NPAD, 1), _f32),
        ],
    )(p_part, d0, d1, bias, Wg_l, ats, atd)


def _kdec_body(p_ref, d0_ref, d1_ref, bias_ref, wdec_ref, bdec_ref, out_ref):
    h = _finish(p_ref, d0_ref, d1_ref, bias_ref)
    out_ref[...] = jnp.dot(h, wdec_ref[...], **_DOT) + bdec_ref[...][None, :]


def _kdec(p_part, d0, d1, bias, W_dec, b_dec):
    blk = 512
    return pl.pallas_call(
        _kdec_body,
        grid=(NPAD // blk,),
        in_specs=[
            pl.BlockSpec((NC, blk, HID), lambda i: (0, i, 0)),
            pl.BlockSpec((blk, 1), lambda i: (i, 0)),
            pl.BlockSpec((blk, 1), lambda i: (i, 0)),
            pl.BlockSpec((HID,), lambda i: (0,)),
            pl.BlockSpec((HID, OUT_F * FH), lambda i: (0, 0)),
            pl.BlockSpec((OUT_F * FH,), lambda i: (0,)),
        ],
        out_specs=pl.BlockSpec((blk, OUT_F * FH), lambda i: (i, 0)),
        out_shape=jax.ShapeDtypeStruct((NPAD, OUT_F * FH), _f32),
    )(p_part, d0, d1, bias, W_dec, b_dec)


def kernel(x, edge_index, edge_attr, W_emb, b_emb, ln_g, ln_b,
           Wg, att_s, att_d, We, att_e, bg, W_dec, b_dec):
    src = edge_index[0]
    dst = edge_index[1]
    npe = EPAD - E
    pad_idx = (jnp.arange(npe, dtype=_i32) % (NPAD - N)) + N
    src_p = jnp.concatenate([src, pad_idx])
    dst_p = jnp.concatenate([dst, pad_idx])
    ea_p = jnp.concatenate([edge_attr, jnp.zeros((npe, ED), _f32)], axis=0)
    x2 = jnp.pad(x.reshape(N, IN_F * INPUT_SIZE), ((0, NPAD - N), (0, 0)))

    z_acc = jnp.zeros((SLC, HID), _f32)
    z_den = jnp.zeros((SLC,), _f32)
    z_ea = jnp.zeros((SLC, ED), _f32)
    qneg = jnp.full((NPAD,), QNEG, _i32)

    sd = jnp.stack([src_p.reshape(EPAD // CHUNK, CHUNK),
                    dst_p.reshape(EPAD // CHUNK, CHUNK)], axis=1)
    deg_part, easum_part = _build_p1()(dst_p, ea_p, z_ea, z_den)
    wae = _waek(We, att_e)
    AE = _aek(ea_p, wae)
    LAE = _laek(easum_part, deg_part, wae)

    H, a_s, a_d = _k0(x2, W_emb, b_emb, ln_g, ln_b,
                      Wg[0], att_s[0][:, None], att_d[0][:, None])
    out = None
    for l in range(L5):
        asrc = a_s[:, 0]
        adst = a_d[:, 0]
        ee, el, qmax_part = _build_sca()(
            sd, AE[l], LAE[l], asrc, adst, qneg)
        qmax = _qmergek(qmax_part)
        p_part, den_part = _build_scbc()(
            sd, ee, el, qmax, H, z_acc, z_den)
        d0 = den_part[0][:, None]
        d1 = den_part[1][:, None]
        if l < L5 - 1:
            H, a_s, a_d = _kl(p_part, d0, d1, bg[l],
                              Wg[l + 1], att_s[l + 1][:, None],
                              att_d[l + 1][:, None])
        else:
            out = _kdec(p_part, d0, d1, bg[l], W_dec, b_dec)
    return out[:N].reshape(N, OUT_F, FH)
